# baseline (speedup 1.0000x reference)
import functools

import jax
import jax.numpy as jnp
from jax.experimental import pallas as pl
from jax.experimental.pallas import tpu as pltpu
from jax.experimental.pallas import tpu_sc as plsc

_B = 16384
_F = 100
_NF = 26
_D = 16
_FD = _F * _D
_BT = 2048
_FP = 128


@functools.partial(
    pl.kernel,
    mesh=plsc.VectorSubcoreMesh(core_axis_name="c", subcore_axis_name="s"),
    out_type=jax.ShapeDtypeStruct((_FP, 128), jnp.float32),
    scratch_types=[
        pltpu.VMEM((8,), jnp.int32),
        pltpu.VMEM((8, 128), jnp.float32),
        pltpu.SemaphoreType.DMA,
    ],
)
def _sc_gather(v_hbm, fi_hbm, out_hbm, idx_v, rows_v, sem):
    wid = jax.lax.axis_index("s") * 2 + jax.lax.axis_index("c")

    @pl.when(wid < _FP // 8)
    def _():
        base = wid * 8
        pltpu.sync_copy(fi_hbm.at[pl.ds(base, 8)], idx_v)
        pltpu.async_copy(v_hbm.at[idx_v], rows_v, sem).wait()
        pltpu.sync_copy(rows_v, out_hbm.at[pl.ds(base, 8)])


def _fm_kernel(x_ref, w_ref, e_ref, yfm_ref, out_ref, pt_ref, a_ref):
    @pl.when(pl.program_id(0) == 0)
    def _init():
        embeds_t = jnp.transpose(e_ref[...], (1, 0))[:_D, :_F]
        j_iota = jax.lax.broadcasted_iota(jnp.int32, (_FD, _D), 0)
        d_iota = jax.lax.broadcasted_iota(jnp.int32, (_FD, _D), 1)
        tm_t = (j_iota % _D == d_iota).astype(jnp.float32)
        emb_rows = jnp.dot(tm_t, embeds_t,
                           preferred_element_type=jnp.float32)
        jf = jax.lax.broadcasted_iota(jnp.int32, (_FD, _F), 0) // _D
        f_iota = jax.lax.broadcasted_iota(jnp.int32, (_FD, _F), 1)
        pt_ref[...] = jnp.where(jf == f_iota, emb_rows, 0.0).astype(jnp.bfloat16)
        esum = jnp.sum(embeds_t, axis=0, keepdims=True)
        esq = jnp.sum(embeds_t * embeds_t, axis=0, keepdims=True)
        a_ref[...] = jnp.concatenate([esum, esq], axis=0)

    xt = x_ref[...]
    out_ref[...] = jnp.dot(pt_ref[...], xt.astype(jnp.bfloat16),
                           preferred_element_type=jnp.float32)
    ws = jnp.concatenate([w_ref[...], a_ref[0:1, :]], axis=0)
    m1 = jnp.dot(ws, xt, preferred_element_type=jnp.float32)
    q = jnp.dot(a_ref[1:2, :], xt * xt,
                preferred_element_type=jnp.float32)
    inter = 0.5 * (m1[1:2, :] * m1[1:2, :] - q)
    yfm_ref[...] = jnp.concatenate([m1[0:1, :], inter], axis=0)


@jax.jit
def kernel(inputs, w, V, field_index):
    w_row = w.reshape(1, _F)
    x_t = inputs.T
    fi_pad = jnp.pad(field_index, (0, _FP - _F))
    v_pad = jnp.pad(V, ((0, 0), (0, 128 - _D)))
    embeds = _sc_gather(v_pad, fi_pad)
    yfm_t, out_t = pl.pallas_call(
        _fm_kernel,
        grid=(_B // _BT,),
        in_specs=[
            pl.BlockSpec((_F, _BT), lambda i: (0, i)),
            pl.BlockSpec((1, _F), lambda i: (0, 0)),
            pl.BlockSpec((_FP, 128), lambda i: (0, 0)),
        ],
        out_specs=[
            pl.BlockSpec((2, _BT), lambda i: (0, i)),
            pl.BlockSpec((_FD, _BT), lambda i: (0, i)),
        ],
        out_shape=[
            jax.ShapeDtypeStruct((2, _B), jnp.float32),
            jax.ShapeDtypeStruct((_FD, _B), jnp.float32),
        ],
        scratch_shapes=[
            pltpu.VMEM((_FD, _F), jnp.bfloat16),
            pltpu.VMEM((2, _F), jnp.float32),
        ],
        compiler_params=pltpu.CompilerParams(
            dimension_semantics=("arbitrary",),
        ),
    )(x_t, w_row, embeds)
    y_fm = yfm_t.T
    new_inputs = out_t.reshape(_F, _D, _B).transpose(2, 0, 1)
    return y_fm, new_inputs


# speedup vs baseline: 1.5325x; 1.5325x over previous
import jax
import jax.numpy as jnp
from jax.experimental import pallas as pl
from jax.experimental.pallas import tpu as pltpu

_B = 16384
_F = 100
_NF = 26
_D = 16
_FD = _F * _D
_BT = 2048


def _fm_kernel(x_ref, w_ref, vt_ref, fi_ref, yfm_ref, out_ref, pt_ref, a_ref):
    @pl.when(pl.program_id(0) == 0)
    def _init():
        fi = fi_ref[...]
        k_iota = jax.lax.broadcasted_iota(jnp.int32, (_NF, _F), 0)
        onehot_t = (fi == k_iota.astype(jnp.float32)).astype(jnp.float32)
        embeds_t = jnp.dot(vt_ref[...], onehot_t,
                           preferred_element_type=jnp.float32)
        j_iota = jax.lax.broadcasted_iota(jnp.int32, (_FD, _D), 0)
        d_iota = jax.lax.broadcasted_iota(jnp.int32, (_FD, _D), 1)
        tm_t = (j_iota % _D == d_iota).astype(jnp.float32)
        emb_rows = jnp.dot(tm_t, embeds_t,
                           preferred_element_type=jnp.float32)
        jf = jax.lax.broadcasted_iota(jnp.int32, (_FD, _F), 0) // _D
        f_iota = jax.lax.broadcasted_iota(jnp.int32, (_FD, _F), 1)
        pt_ref[...] = jnp.where(jf == f_iota, emb_rows, 0.0).astype(jnp.bfloat16)
        esum = jnp.sum(embeds_t, axis=0, keepdims=True)
        esq = jnp.sum(embeds_t * embeds_t, axis=0, keepdims=True)
        a_ref[...] = jnp.concatenate([esum, esq], axis=0)

    xt = x_ref[...]
    out_ref[...] = jnp.dot(pt_ref[...], xt.astype(jnp.bfloat16),
                           preferred_element_type=jnp.float32)
    ws = jnp.concatenate([w_ref[...], a_ref[0:1, :]], axis=0)
    m1 = jnp.dot(ws, xt, preferred_element_type=jnp.float32)
    q = jnp.dot(a_ref[1:2, :], xt * xt,
                preferred_element_type=jnp.float32)
    inter = 0.5 * (m1[1:2, :] * m1[1:2, :] - q)
    yfm_ref[...] = jnp.concatenate([m1[0:1, :], inter], axis=0)


@jax.jit
def kernel(inputs, w, V, field_index):
    fi_row = field_index.astype(jnp.float32).reshape(1, _F)
    w_row = w.reshape(1, _F)
    v_t = V.T
    x_t = inputs.T
    yfm_t, out_t = pl.pallas_call(
        _fm_kernel,
        grid=(_B // _BT,),
        in_specs=[
            pl.BlockSpec((_F, _BT), lambda i: (0, i)),
            pl.BlockSpec((1, _F), lambda i: (0, 0)),
            pl.BlockSpec((_D, _NF), lambda i: (0, 0)),
            pl.BlockSpec((1, _F), lambda i: (0, 0)),
        ],
        out_specs=[
            pl.BlockSpec((2, _BT), lambda i: (0, i)),
            pl.BlockSpec((_FD, _BT), lambda i: (0, i)),
        ],
        out_shape=[
            jax.ShapeDtypeStruct((2, _B), jnp.float32),
            jax.ShapeDtypeStruct((_FD, _B), jnp.float32),
        ],
        scratch_shapes=[
            pltpu.VMEM((_FD, _F), jnp.bfloat16),
            pltpu.VMEM((2, _F), jnp.float32),
        ],
        compiler_params=pltpu.CompilerParams(
            dimension_semantics=("arbitrary",),
        ),
    )(x_t, w_row, v_t, fi_row)
    y_fm = yfm_t.T
    new_inputs = out_t.reshape(_F, _D, _B).transpose(2, 0, 1)
    return y_fm, new_inputs
